# Initial kernel scaffold; baseline (speedup 1.0000x reference)
import jax
import jax.numpy as jnp
from jax.experimental import pallas as pl


def kernel(x, edge_index, W, B):
    raise NotImplementedError("write your pallas kernel here")


# speedup vs baseline: 7.8359x; 7.8359x over previous
import functools

import jax
import jax.numpy as jnp
from jax import lax
from jax.experimental import pallas as pl
from jax.experimental.pallas import tpu as pltpu
from jax.experimental.pallas import tpu_sc as plsc

N_NODES = 10000
N_EDGES = 320000
D = 128

NC = 2
NS = 16
NW = NC * NS

EPW = N_EDGES // NW
CHUNK = 80
NCHK = EPW // CHUNK
N_PAD = 10240
RPT = N_PAD // NS


def _sc_body(x_hbm, src_hbm, dst_hbm, z2d_hbm, z1d_hbm, agg_out, deg_out,
             idx_s, idx_d, rows, ones_v, sem_d, agg_sh, deg_sh):
  cid = lax.axis_index("c")
  sid = lax.axis_index("s")
  wid = sid * NC + cid

  for i in range(CHUNK // 16):
    ones_v[pl.ds(i * 16, 16)] = jnp.full((16,), 1.0, jnp.float32)

  r0 = sid * RPT
  pltpu.sync_copy(z2d_hbm.at[pl.ds(r0, RPT)], agg_sh.at[pl.ds(r0, RPT)])
  pltpu.sync_copy(z1d_hbm.at[pl.ds(r0, RPT)], deg_sh.at[pl.ds(r0, RPT)])

  pltpu.sync_copy(src_hbm.at[wid], idx_s)
  pltpu.sync_copy(dst_hbm.at[wid], idx_d)

  plsc.subcore_barrier()

  def chunk_body(c, carry):
    dcp = pltpu.async_copy(ones_v, deg_sh.at[idx_d.at[c]], sem_d, add=True)
    pltpu.sync_copy(x_hbm.at[idx_s.at[c]], rows)
    pltpu.sync_copy(rows, agg_sh.at[idx_d.at[c]], add=True)
    dcp.wait()
    return carry

  lax.fori_loop(0, NCHK, chunk_body, 0)

  plsc.subcore_barrier()

  pltpu.sync_copy(agg_sh.at[pl.ds(r0, RPT)], agg_out.at[cid, pl.ds(r0, RPT)])
  pltpu.sync_copy(deg_sh.at[pl.ds(r0, RPT)], deg_out.at[cid, pl.ds(r0, RPT)])


@jax.jit
def _sc_accumulate(x, src3, dst3, z2d, z1d):
  mesh = plsc.VectorSubcoreMesh(
      core_axis_name="c", subcore_axis_name="s", num_cores=NC, num_subcores=NS)
  kern = pl.kernel(
      _sc_body,
      out_type=[
          jax.ShapeDtypeStruct((NC, N_PAD, D), jnp.float32),
          jax.ShapeDtypeStruct((NC, N_PAD), jnp.float32),
      ],
      mesh=mesh,
      scratch_types=[
          pltpu.VMEM((NCHK, CHUNK), jnp.int32),
          pltpu.VMEM((NCHK, CHUNK), jnp.int32),
          pltpu.VMEM((CHUNK, D), jnp.float32),
          pltpu.VMEM((CHUNK,), jnp.float32),
          pltpu.SemaphoreType.DMA,
          pltpu.VMEM_SHARED((N_PAD, D), jnp.float32),
          pltpu.VMEM_SHARED((N_PAD,), jnp.float32),
      ],
  )
  return kern(x, src3, dst3, z2d, z1d)


RB = 1024


def _tc_body(agg_ref, deg_ref, x_ref, w_ref, b_ref, o_ref):
  agg = agg_ref[0] + agg_ref[1]
  deg = jnp.maximum(deg_ref[0] + deg_ref[1], 1.0)
  normed = agg / deg
  dn = (((1,), (1,)), ((), ()))
  o_ref[...] = (
      lax.dot_general(normed, w_ref[...], dn,
                      preferred_element_type=jnp.float32)
      + lax.dot_general(x_ref[...], b_ref[...], dn,
                        preferred_element_type=jnp.float32))


@jax.jit
def _tc_finish(agg_p, deg_p, xp, W, B):
  grid = N_PAD // RB
  deg3 = deg_p.reshape(NC, N_PAD, 1)
  return pl.pallas_call(
      _tc_body,
      grid=(grid,),
      in_specs=[
          pl.BlockSpec((NC, RB, D), lambda i: (0, i, 0)),
          pl.BlockSpec((NC, RB, 1), lambda i: (0, i, 0)),
          pl.BlockSpec((RB, D), lambda i: (i, 0)),
          pl.BlockSpec((D, D), lambda i: (0, 0)),
          pl.BlockSpec((D, D), lambda i: (0, 0)),
      ],
      out_specs=pl.BlockSpec((RB, D), lambda i: (i, 0)),
      out_shape=jax.ShapeDtypeStruct((N_PAD, D), jnp.float32),
  )(agg_p, deg3, xp, W, B)


def kernel(x, edge_index, W, B):
  src3 = edge_index[0].reshape(NW, NCHK, CHUNK)
  dst3 = edge_index[1].reshape(NW, NCHK, CHUNK)
  z2d = jnp.zeros((N_PAD, D), jnp.float32)
  z1d = jnp.zeros((N_PAD,), jnp.float32)
  agg_p, deg_p = _sc_accumulate(x, src3, dst3, z2d, z1d)
  xp = jnp.pad(x, ((0, N_PAD - N_NODES), (0, 0)))
  out = _tc_finish(agg_p, deg_p, xp, W, B)
  return out[:N_NODES]


# speedup vs baseline: 9.9689x; 1.2722x over previous
import jax
import jax.numpy as jnp
from jax import lax
from jax.experimental import pallas as pl
from jax.experimental.pallas import tpu as pltpu
from jax.experimental.pallas import tpu_sc as plsc

N_NODES = 10000
N_EDGES = 320000
D = 128

NC = 2
NS = 16
NW = NC * NS

EPW = N_EDGES // NW
CHUNK = 40
NCHK = EPW // CHUNK
N_PAD = 10240
RPT = N_PAD // NS
BUF = CHUNK
NWB = RPT // BUF
ONES_LEN = 48


def _sc_body(x_hbm, src_hbm, dst_hbm, z2d_hbm, z1d_hbm, agg_out, deg_out,
             agg_sh, deg_sh, idx_s, idx_d, rows_a, rows_b, ones_v,
             sem_a, sem_b, sem_d):
  cid = lax.axis_index("c")
  sid = lax.axis_index("s")
  wid = sid * NC + cid

  for i in range(ONES_LEN // 16):
    ones_v[pl.ds(i * 16, 16)] = jnp.full((16,), 1.0, jnp.float32)

  r0 = sid * RPT
  pltpu.sync_copy(z2d_hbm, rows_a)
  for j in range(NWB):
    pltpu.sync_copy(rows_a, agg_sh.at[pl.ds(r0 + j * BUF, BUF)])
  pltpu.sync_copy(z1d_hbm.at[pl.ds(r0, RPT)], deg_sh.at[pl.ds(r0, RPT)])

  pltpu.sync_copy(src_hbm.at[wid], idx_s)
  pltpu.sync_copy(dst_hbm.at[wid], idx_d)

  plsc.subcore_barrier()

  buf_a = rows_a
  buf_b = rows_b
  last = NCHK - 1
  def s_at(k):
    return idx_s.at[pl.ds(k * CHUNK, CHUNK)]

  def d_at(k):
    return idx_d.at[pl.ds(k * CHUNK, CHUNK)]

  pltpu.async_copy(x_hbm.at[s_at(0)], buf_a, sem_a)
  pltpu.async_copy(x_hbm.at[s_at(1)], buf_b, sem_b)

  def pair_body(i, carry):
    k = 2 * i
    d0 = pltpu.async_copy(ones_v.at[pl.ds(0, CHUNK)],
                          deg_sh.at[d_at(k)], sem_d, add=True)
    d1 = pltpu.async_copy(ones_v.at[pl.ds(0, CHUNK)],
                          deg_sh.at[d_at(k + 1)], sem_d, add=True)
    pltpu.make_async_copy(x_hbm.at[s_at(k)], buf_a, sem_a).wait()
    pltpu.sync_copy(buf_a, agg_sh.at[d_at(k)], add=True)
    ka = jnp.minimum(k + 2, last)
    pltpu.async_copy(x_hbm.at[s_at(ka)], buf_a, sem_a)
    pltpu.make_async_copy(x_hbm.at[s_at(k + 1)], buf_b, sem_b).wait()
    pltpu.sync_copy(buf_b, agg_sh.at[d_at(k + 1)], add=True)
    kb = jnp.minimum(k + 3, last)
    pltpu.async_copy(x_hbm.at[s_at(kb)], buf_b, sem_b)
    d0.wait()
    d1.wait()
    return carry

  lax.fori_loop(0, NCHK // 2, pair_body, 0)

  pltpu.make_async_copy(x_hbm.at[s_at(last)], buf_a, sem_a).wait()
  pltpu.make_async_copy(x_hbm.at[s_at(last)], buf_b, sem_b).wait()

  plsc.subcore_barrier()

  bufs = (rows_a, rows_b)
  sems = (sem_a, sem_b)
  for j in range(NWB):
    buf, sem = bufs[j % 2], sems[j % 2]
    dst_prev = agg_out.at[cid, pl.ds(r0 + (j - 2) * BUF, BUF)]
    if j >= 2:
      pltpu.make_async_copy(bufs[j % 2], dst_prev, sem).wait()
    pltpu.sync_copy(agg_sh.at[pl.ds(r0 + j * BUF, BUF)], buf)
    pltpu.async_copy(buf, agg_out.at[cid, pl.ds(r0 + j * BUF, BUF)], sem)
  for j in (NWB - 2, NWB - 1):
    buf, sem = bufs[j % 2], sems[j % 2]
    pltpu.make_async_copy(
        buf, agg_out.at[cid, pl.ds(r0 + j * BUF, BUF)], sem).wait()
  pltpu.sync_copy(deg_sh.at[pl.ds(r0, RPT)], deg_out.at[cid, pl.ds(r0, RPT)])


def _sc_accumulate(x, src3, dst3, z2d, z1d):
  mesh = plsc.VectorSubcoreMesh(
      core_axis_name="c", subcore_axis_name="s", num_cores=NC, num_subcores=NS)
  kern = pl.kernel(
      _sc_body,
      out_type=[
          jax.ShapeDtypeStruct((NC, N_PAD, D), jnp.float32),
          jax.ShapeDtypeStruct((NC, N_PAD), jnp.float32),
      ],
      mesh=mesh,
      scratch_types=[
          pltpu.VMEM_SHARED((N_PAD, D), jnp.float32),
          pltpu.VMEM_SHARED((N_PAD,), jnp.float32),
          pltpu.VMEM((EPW,), jnp.int32),
          pltpu.VMEM((EPW,), jnp.int32),
          pltpu.VMEM((BUF, D), jnp.float32),
          pltpu.VMEM((BUF, D), jnp.float32),
          pltpu.VMEM((ONES_LEN,), jnp.float32),
          pltpu.SemaphoreType.DMA,
          pltpu.SemaphoreType.DMA,
          pltpu.SemaphoreType.DMA,
      ],
  )
  return kern(x, src3, dst3, z2d, z1d)


RB = 1000


def _tc_body(agg_ref, deg_ref, x_ref, w_ref, b_ref, o_ref):
  agg = agg_ref[0] + agg_ref[1]
  deg = jnp.maximum(deg_ref[0] + deg_ref[1], 1.0)
  normed = agg / deg
  dn = (((1,), (1,)), ((), ()))
  o_ref[...] = (
      lax.dot_general(normed, w_ref[...], dn,
                      preferred_element_type=jnp.float32)
      + lax.dot_general(x_ref[...], b_ref[...], dn,
                        preferred_element_type=jnp.float32))


def _tc_finish(agg_p, deg_p, x, W, B):
  grid = N_NODES // RB
  deg3 = deg_p.reshape(NC, N_PAD, 1)
  return pl.pallas_call(
      _tc_body,
      grid=(grid,),
      in_specs=[
          pl.BlockSpec((NC, RB, D), lambda i: (0, i, 0)),
          pl.BlockSpec((NC, RB, 1), lambda i: (0, i, 0)),
          pl.BlockSpec((RB, D), lambda i: (i, 0)),
          pl.BlockSpec((D, D), lambda i: (0, 0)),
          pl.BlockSpec((D, D), lambda i: (0, 0)),
      ],
      out_specs=pl.BlockSpec((RB, D), lambda i: (i, 0)),
      out_shape=jax.ShapeDtypeStruct((N_NODES, D), jnp.float32),
  )(agg_p, deg3, x, W, B)


def kernel(x, edge_index, W, B):
  src3 = edge_index[0].reshape(NW, EPW)
  dst3 = edge_index[1].reshape(NW, EPW)
  z2d = jnp.zeros((BUF, D), jnp.float32)
  z1d = jnp.zeros((N_PAD,), jnp.float32)
  agg_p, deg_p = _sc_accumulate(x, src3, dst3, z2d, z1d)
  return _tc_finish(agg_p, deg_p, x, W, B)
